# baseline (speedup 1.0000x reference)
import jax
import jax.numpy as jnp
from jax import lax
from jax.experimental import pallas as pl
from jax.experimental.pallas import tpu as pltpu
from jax.experimental.pallas import tpu_sc as plsc

B, H, W, C = 8, 384, 384, 96
HW = H * W
NC, NS = 2, 16

CG = 8
HS = 128
KH = 8
TG = H // HS


def _tc_body_factory(b_off):
    def _tc_body(x_ref, o_ref, vscr, iscr):
        t = pl.program_id(2)

        @pl.when(t == 0)
        def _():
            vscr[...] = jnp.full((KH, CG, W), -jnp.inf, jnp.float32)
            iscr[...] = jnp.zeros((KH, CG, W), jnp.int32)

        rv = vscr[...]
        ri = iscr[...]
        for i in range(HS // KH):
            s = t * (HS // KH) + i
            chunk = x_ref[0, pl.ds(i * KH, KH), :, :]
            m = chunk > rv
            rv = jnp.maximum(chunk, rv)
            ri = jnp.where(m, s, ri)
        vscr[...] = rv
        iscr[...] = ri

        @pl.when(t == TG - 1)
        def _():
            chain = jax.lax.broadcasted_iota(jnp.int32, (KH, CG, W), 0)
            wlane = jax.lax.broadcasted_iota(jnp.int32, (KH, CG, W), 2)
            fl = (ri * KH + chain) * W + wlane
            fv = jnp.max(rv, axis=(0, 2))
            win = rv == fv[None, :, None]
            bf = jnp.min(jnp.where(win, fl, HW), axis=(0, 2))
            y = bf // W
            x = bf - y * W
            o_ref[0, 0, 0, :] = y.astype(jnp.float32)
            o_ref[0, 0, 1, :] = x.astype(jnp.float32)
    return _tc_body


def _tc_run(xt, b_off, nb):
    out = pl.pallas_call(
        _tc_body_factory(b_off),
        grid=(nb, C // CG, TG),
        in_specs=[pl.BlockSpec((1, HS, CG, W),
                               lambda b, cg, t: (b + b_off, t, cg, 0))],
        out_specs=pl.BlockSpec((1, 1, 2, CG), lambda b, cg, t: (b, cg, 0, 0)),
        out_shape=jax.ShapeDtypeStruct((nb, C // CG, 2, CG), jnp.float32),
        scratch_shapes=[
            pltpu.VMEM((KH, CG, W), jnp.float32),
            pltpu.VMEM((KH, CG, W), jnp.int32),
        ],
        compiler_params=pltpu.CompilerParams(
            dimension_semantics=("parallel", "parallel", "arbitrary"),
        ),
    )(xt)
    return jnp.reshape(jnp.transpose(out, (0, 2, 1, 3)), (nb, 2, C))


RWS = HW // (NC * NS)
RS = 256
CHS = RS * C
NCHUNKS = RWS // RS
NBS = 2
G = C // 16


def _sc_body(x_hbm, out_hbm, *scr):
    bufs = scr[:NBS]
    pval, pidx, shval, shidx, cmbv, cmbi, obuf = scr[NBS:NBS + 7]
    sems = scr[NBS + 7:]
    c = lax.axis_index("c")
    s = lax.axis_index("s")
    rank = c * NS + s
    spat_base = rank * RWS
    base_off = spat_base * C

    def start(k, j):
        off = base_off + k * CHS
        pltpu.make_async_copy(x_hbm.at[pl.ds(off, CHS)], bufs[j], sems[j]).start()

    def wait(j):
        pltpu.make_async_copy(x_hbm.at[pl.ds(0, CHS)], bufs[j], sems[j]).wait()

    for j in range(NBS):
        start(j, j)

    ninf = jnp.full((16,), -jnp.inf, jnp.float32)
    zero = jnp.zeros((16,), jnp.int32)
    state = tuple([ninf] * G + [zero] * G)

    def chunk_rows(buf):
        def row_body(r, carry):
            rvec = carry[0]
            bvs = list(carry[1:1 + G])
            bis = list(carry[1 + G:])
            base = r * C
            for g in range(G):
                v = buf[pl.ds(base + g * 16, 16)]
                m = v > bvs[g]
                bvs[g] = jnp.where(m, v, bvs[g])
                bis[g] = jnp.where(m, rvec, bis[g])
            return (rvec + 1,) + tuple(bvs) + tuple(bis)
        return row_body

    def outer(t, state):
        for j in range(NBS):
            k = t * NBS + j
            wait(j)
            rvec0 = zero + (spat_base + k * RS)
            carry = (rvec0,) + state
            carry = lax.fori_loop(0, RS, chunk_rows(bufs[j]), carry, unroll=2)
            state = carry[1:]

            @pl.when(k + NBS < NCHUNKS)
            def _():
                start(k + NBS, j)
        return state

    state = lax.fori_loop(0, NCHUNKS // NBS, outer, state)
    bvs = state[:G]
    bis = state[G:]

    for g in range(G):
        pval[pl.ds(g * 16, 16)] = bvs[g]
        pidx[pl.ds(g * 16, 16)] = bis[g]
    pltpu.sync_copy(pval, shval.at[pl.ds(s * C, C)])
    pltpu.sync_copy(pidx, shidx.at[pl.ds(s * C, C)])
    plsc.subcore_barrier()

    @pl.when(s == 0)
    def _():
        pltpu.sync_copy(shval, cmbv)
        pltpu.sync_copy(shidx, cmbi)
        for g in range(G):
            bv = cmbv[pl.ds(g * 16, 16)]
            bi = cmbi[pl.ds(g * 16, 16)]
            for j in range(1, NS):
                v = cmbv[pl.ds(j * C + g * 16, 16)]
                i = cmbi[pl.ds(j * C + g * 16, 16)]
                m = v > bv
                bv = jnp.where(m, v, bv)
                bi = jnp.where(m, i, bi)
            t = lax.shift_right_logical(bi, 7)
            y = lax.shift_right_logical(t * 21846, 16)
            x = bi - y * W
            obuf[pl.ds(g * 16, 16)] = y.astype(jnp.float32)
            obuf[pl.ds(C + g * 16, 16)] = x.astype(jnp.float32)
            obuf[pl.ds(2 * C + g * 16, 16)] = bv
        pltpu.sync_copy(obuf, out_hbm.at[c])


def _sc_run(flat0):
    run = pl.kernel(
        _sc_body,
        out_type=jax.ShapeDtypeStruct((NC, 3 * C), jnp.float32),
        mesh=plsc.VectorSubcoreMesh(core_axis_name="c", subcore_axis_name="s"),
        scratch_types=(
            [pltpu.VMEM((CHS,), jnp.float32) for _ in range(NBS)]
            + [pltpu.VMEM((C,), jnp.float32), pltpu.VMEM((C,), jnp.int32),
               pltpu.VMEM_SHARED((NS * C,), jnp.float32),
               pltpu.VMEM_SHARED((NS * C,), jnp.int32),
               pltpu.VMEM((NS * C,), jnp.float32),
               pltpu.VMEM((NS * C,), jnp.int32),
               pltpu.VMEM((3 * C,), jnp.float32)]
            + [pltpu.SemaphoreType.DMA for _ in range(NBS)]
        ),
    )
    return run(flat0)


@jax.jit
def kernel(inputs):
    xt = jnp.transpose(inputs, (0, 1, 3, 2))
    flat0 = jnp.reshape(inputs[0], (HW * C,))
    sc = _sc_run(flat0)
    tc_a = _tc_run(xt, 1, 2)
    tc_b = _tc_run(xt, 3, 5)
    v0, v1 = sc[0, 2 * C:], sc[1, 2 * C:]
    m = v1 > v0
    y = jnp.where(m, sc[1, :C], sc[0, :C])
    x = jnp.where(m, sc[1, C:2 * C], sc[0, C:2 * C])
    b0 = jnp.stack([y, x], axis=0)[None]
    return jnp.concatenate([b0, tc_a, tc_b], axis=0)


# speedup vs baseline: 1.5522x; 1.5522x over previous
import jax
import jax.numpy as jnp
from jax import lax
from jax.experimental import pallas as pl
from jax.experimental.pallas import tpu as pltpu

B, H, W, C = 8, 384, 384, 96
HW = H * W
CG = 8
HS = 128
KH = 8
TG = H // HS


def _argmax_body(x_ref, o_ref, vscr, iscr):
    t = pl.program_id(2)

    @pl.when(t == 0)
    def _():
        vscr[...] = jnp.full((KH, CG, W), -jnp.inf, jnp.float32)
        iscr[...] = jnp.zeros((KH, CG, W), jnp.int32)

    rv = vscr[...]
    ri = iscr[...]
    for i in range(HS // KH):
        s = t * (HS // KH) + i
        chunk = x_ref[0, pl.ds(i * KH, KH), :, :]
        m = chunk > rv
        rv = jnp.maximum(chunk, rv)
        ri = jnp.where(m, s, ri)
    vscr[...] = rv
    iscr[...] = ri

    @pl.when(t == TG - 1)
    def _():
        chain = jax.lax.broadcasted_iota(jnp.int32, (KH, CG, W), 0)
        wlane = jax.lax.broadcasted_iota(jnp.int32, (KH, CG, W), 2)
        fl = (ri * KH + chain) * W + wlane
        fv = jnp.max(rv, axis=(0, 2))
        win = rv == fv[None, :, None]
        bf = jnp.min(jnp.where(win, fl, HW), axis=(0, 2))
        y = bf // W
        x = bf - y * W
        o_ref[0, 0, 0, :] = y.astype(jnp.float32)
        o_ref[0, 0, 1, :] = x.astype(jnp.float32)


@jax.jit
def kernel(inputs):
    xt = jnp.transpose(inputs, (0, 1, 3, 2))
    out = pl.pallas_call(
        _argmax_body,
        grid=(B, C // CG, TG),
        in_specs=[pl.BlockSpec((1, HS, CG, W), lambda b, cg, t: (b, t, cg, 0))],
        out_specs=pl.BlockSpec((1, 1, 2, CG), lambda b, cg, t: (b, cg, 0, 0)),
        out_shape=jax.ShapeDtypeStruct((B, C // CG, 2, CG), jnp.float32),
        scratch_shapes=[
            pltpu.VMEM((KH, CG, W), jnp.float32),
            pltpu.VMEM((KH, CG, W), jnp.int32),
        ],
        compiler_params=pltpu.CompilerParams(
            dimension_semantics=("parallel", "parallel", "arbitrary"),
        ),
    )(xt)
    return jnp.reshape(jnp.transpose(out, (0, 2, 1, 3)), (B, 2, C))
